# baseline (speedup 1.0000x reference)
import functools

import jax
import jax.numpy as jnp
from jax import lax
from jax.experimental import pallas as pl
from jax.experimental.pallas import tpu as pltpu
from jax.experimental.pallas import tpu_sc as plsc

N = 10000
E = 320000
D = 128

NC = 2
NS = 16
NW = NC * NS
CL = 128

NP = 10240
RPC = NP // NS
GPW = NP // NW
LG = 64
YC = GPW // LG

EPT = 10240
EC = EPT // CL
PH = 2
HC = EC // PH
DUMP = NP - 8

def _sc_embed_deg(embed_hbm, yb_hbm, dstb_hbm, ones_hbm, ztile_hbm,
                  e_out, deg_out, y_v, dst_v, rows_v, ones_v, deg_sh, sem):
    c = lax.axis_index("c")
    s = lax.axis_index("s")
    wid = c * NS + s
    pltpu.sync_copy(yb_hbm.at[wid], y_v)
    pltpu.sync_copy(dstb_hbm.at[wid], dst_v)
    pltpu.sync_copy(ones_hbm, ones_v)
    pltpu.sync_copy(ztile_hbm, deg_sh.at[pl.ds(s * RPC, RPC)])
    plsc.subcore_barrier()

    def gbody(j, _):
        pltpu.async_copy(embed_hbm.at[y_v.at[j]], rows_v, sem).wait()
        pltpu.sync_copy(rows_v, e_out.at[pl.ds(wid * GPW + j * LG, LG)])
        return 0

    lax.fori_loop(0, YC, gbody, 0)

    def dbody(j, _):
        pltpu.sync_copy(ones_v, deg_sh.at[dst_v.at[j]], add=True)
        return 0

    lax.fori_loop(0, EC, dbody, 0)
    plsc.subcore_barrier()
    pltpu.sync_copy(deg_sh.at[pl.ds(s * RPC, RPC)],
                    deg_out.at[c, pl.ds(s * RPC, RPC)])


def _sc_edge_agg(xs_hbm, srcb_hbm, dstb_hbm, ztile_hbm, acc_out,
                 src_v, dst_v, buf0, buf1, acc_sh, sem0, sem1):
    c = lax.axis_index("c")
    s = lax.axis_index("s")
    wid = c * NS + s
    pltpu.sync_copy(ztile_hbm, acc_sh.at[pl.ds(s * RPC, RPC)])
    plsc.subcore_barrier()

    for p in range(PH):
        pltpu.sync_copy(srcb_hbm.at[wid, pl.ds(p * HC, HC)], src_v)
        pltpu.sync_copy(dstb_hbm.at[wid, pl.ds(p * HC, HC)], dst_v)

        pltpu.async_copy(xs_hbm.at[src_v.at[0]], buf0, sem0)
        pltpu.async_copy(xs_hbm.at[src_v.at[1]], buf1, sem1)

        def body(k, _):
            j0 = 2 * k
            pltpu.make_async_copy(xs_hbm.at[src_v.at[j0]], buf0, sem0).wait()
            pltpu.sync_copy(buf0, acc_sh.at[dst_v.at[j0]], add=True)
            pltpu.async_copy(xs_hbm.at[src_v.at[j0 + 2]], buf0, sem0)
            pltpu.make_async_copy(
                xs_hbm.at[src_v.at[j0 + 1]], buf1, sem1).wait()
            pltpu.sync_copy(buf1, acc_sh.at[dst_v.at[j0 + 1]], add=True)
            pltpu.async_copy(xs_hbm.at[src_v.at[j0 + 3]], buf1, sem1)
            return 0

        lax.fori_loop(0, HC // 2 - 1, body, 0)
        pltpu.make_async_copy(xs_hbm.at[src_v.at[HC - 2]], buf0, sem0).wait()
        pltpu.sync_copy(buf0, acc_sh.at[dst_v.at[HC - 2]], add=True)
        pltpu.make_async_copy(xs_hbm.at[src_v.at[HC - 1]], buf1, sem1).wait()
        pltpu.sync_copy(buf1, acc_sh.at[dst_v.at[HC - 1]], add=True)

    plsc.subcore_barrier()
    pltpu.sync_copy(acc_sh.at[pl.ds(s * RPC, RPC)],
                    acc_out.at[c, pl.ds(s * RPC, RPC)])


@functools.lru_cache(maxsize=None)
def _sc_kernels():
    mesh = plsc.VectorSubcoreMesh(core_axis_name="c", subcore_axis_name="s",
                                  num_cores=NC, num_subcores=NS)
    embed_deg = pl.kernel(
        _sc_embed_deg,
        out_type=(
            jax.ShapeDtypeStruct((NP, D), jnp.float32),
            jax.ShapeDtypeStruct((NC, NP, D), jnp.float32),
        ),
        mesh=mesh,
        scratch_types=[
            pltpu.VMEM((YC, LG), jnp.int32),
            pltpu.VMEM((EC, CL), jnp.int32),
            pltpu.VMEM((LG, D), jnp.float32),
            pltpu.VMEM((CL, D), jnp.float32),
            pltpu.VMEM_SHARED((NP, D), jnp.float32),
            pltpu.SemaphoreType.DMA,
        ],
    )
    edge_agg = pl.kernel(
        _sc_edge_agg,
        out_type=jax.ShapeDtypeStruct((NC, NP, D), jnp.float32),
        mesh=mesh,
        scratch_types=[
            pltpu.VMEM((HC, CL), jnp.int32),
            pltpu.VMEM((HC, CL), jnp.int32),
            pltpu.VMEM((CL, D), jnp.float32),
            pltpu.VMEM((CL, D), jnp.float32),
            pltpu.VMEM_SHARED((NP, D), jnp.float32),
            pltpu.SemaphoreType.DMA,
            pltpu.SemaphoreType.DMA,
        ],
    )
    return embed_deg, edge_agg


_BR = 2560


def _tc1_body(d0, d1, e, wn, bn, w1, xs1_ref, dinv_ref):
    dinv = lax.rsqrt(d0[...] + d1[...] + 1.0)
    x0 = jnp.dot(e[...], wn[...], preferred_element_type=jnp.float32) + bn[...]
    xw1 = jnp.dot(x0, w1[...], preferred_element_type=jnp.float32)
    xs1_ref[...] = dinv * xw1
    dinv_ref[...] = dinv


def _tc2_body(a0, a1, xs1, dinv, b1, w2, xs2_ref):
    h = jnp.maximum(dinv[...] * (a0[...] + a1[...] + xs1[...]) + b1[...], 0.0)
    xs2_ref[...] = dinv[...] * jnp.dot(h, w2[...],
                                       preferred_element_type=jnp.float32)


def _tc3_body(a0, a1, xs2, dinv, b2, out_ref):
    out_ref[...] = dinv[...] * (a0[...] + a1[...] + xs2[...]) + b2[...]


def _rows(shape):
    return pl.BlockSpec(shape, lambda i: (i,) + (0,) * (len(shape) - 1))


def _whole(shape):
    return pl.BlockSpec(shape, lambda i: (0,) * len(shape))


def _tc1(d0, d1, e, wn, bn, w1):
    return pl.pallas_call(
        _tc1_body,
        grid=(NP // _BR,),
        in_specs=[_rows((_BR, D)), _rows((_BR, D)), _rows((_BR, D)),
                  _whole((D, D)), _whole((1, D)), _whole((D, D))],
        out_specs=[_rows((_BR, D)), _rows((_BR, D))],
        out_shape=[jax.ShapeDtypeStruct((NP, D), jnp.float32),
                   jax.ShapeDtypeStruct((NP, D), jnp.float32)],
    )(d0, d1, e, wn, bn, w1)


def _tc2(a0, a1, xs1, dinv, b1, w2):
    return pl.pallas_call(
        _tc2_body,
        grid=(NP // _BR,),
        in_specs=[_rows((_BR, D)), _rows((_BR, D)), _rows((_BR, D)),
                  _rows((_BR, D)), _whole((1, D)), _whole((D, D))],
        out_specs=_rows((_BR, D)),
        out_shape=jax.ShapeDtypeStruct((NP, D), jnp.float32),
    )(a0, a1, xs1, dinv, b1, w2)


def _tc3(a0, a1, xs2, dinv, b2):
    return pl.pallas_call(
        _tc3_body,
        grid=(NP // _BR,),
        in_specs=[_rows((_BR, D)), _rows((_BR, D)), _rows((_BR, D)),
                  _rows((_BR, D)), _whole((1, D))],
        out_specs=_rows((_BR, D)),
        out_shape=jax.ShapeDtypeStruct((NP, D), jnp.float32),
    )(a0, a1, xs2, dinv, b2)


def kernel(y, edge_index, embed_table, W_node_w, W_node_b, W1, b1, W2, b2):
    src = edge_index[0]
    dst = edge_index[1]
    epad = NW * EPT - E
    srcb = jnp.concatenate(
        [src, jnp.zeros((epad,), jnp.int32)]).reshape(NW, EC, CL)
    srcb = srcb + jnp.where(
        jnp.arange(NW, dtype=jnp.int32)[:, None, None] >= NS, NP, 0)
    dstb = jnp.concatenate(
        [dst, jnp.full((epad,), DUMP, jnp.int32)]).reshape(NW, EC, CL)
    yb = jnp.concatenate(
        [y, jnp.zeros((NP - N,), jnp.int32)]).reshape(NW, YC, LG)

    ones_rows = jnp.ones((CL, D), jnp.float32)
    ztile = jnp.zeros((RPC, D), jnp.float32)

    sc_embed_deg, sc_edge_agg = _sc_kernels()
    e_rows, deg_p = sc_embed_deg(embed_table, yb, dstb, ones_rows, ztile)
    xs1, dinv = _tc1(deg_p[0], deg_p[1], e_rows,
                     W_node_w, W_node_b.reshape(1, D), W1)
    acc1 = sc_edge_agg(jnp.concatenate([xs1, xs1]), srcb, dstb, ztile)
    xs2 = _tc2(acc1[0], acc1[1], xs1, dinv, b1.reshape(1, D), W2)
    acc2 = sc_edge_agg(jnp.concatenate([xs2, xs2]), srcb, dstb, ztile)
    out = _tc3(acc2[0], acc2[1], xs2, dinv, b2.reshape(1, D))
    return out[:N]


# speedup vs baseline: 1.1219x; 1.1219x over previous
import functools

import jax
import jax.numpy as jnp
from jax import lax
from jax.experimental import pallas as pl
from jax.experimental.pallas import tpu as pltpu
from jax.experimental.pallas import tpu_sc as plsc

N = 10000
E = 320000
D = 128

NC = 2
NS = 16
NW = NC * NS
CL = 128

NP = 10240
RPC = NP // NS
GPW = NP // NW
LG = 64
YC = GPW // LG

EPT = 10240
EC = EPT // CL
PH = 2
HC = EC // PH
DUMP = NP - 8


def _sc_embed_deg(embed_hbm, yb_hbm, dstb_hbm, ones_hbm, ztile_hbm,
                  e_out, deg_out, y_v, dst_v, rows_v, ones_v, deg_sh, sem):
    c = lax.axis_index("c")
    s = lax.axis_index("s")
    wid = c * NS + s
    pltpu.sync_copy(yb_hbm.at[wid], y_v)
    pltpu.sync_copy(dstb_hbm.at[wid], dst_v)
    pltpu.sync_copy(ones_hbm, ones_v)
    pltpu.sync_copy(ztile_hbm, deg_sh.at[pl.ds(s * RPC, RPC)])
    plsc.subcore_barrier()

    def gbody(j, _):
        pltpu.async_copy(embed_hbm.at[y_v.at[j]], rows_v, sem).wait()
        pltpu.sync_copy(rows_v, e_out.at[pl.ds(wid * GPW + j * LG, LG)])
        return 0

    lax.fori_loop(0, YC, gbody, 0)

    def dbody(j, _):
        pltpu.sync_copy(ones_v, deg_sh.at[dst_v.at[j]], add=True)
        return 0

    lax.fori_loop(0, EC, dbody, 0)
    plsc.subcore_barrier()
    pltpu.sync_copy(deg_sh.at[pl.ds(s * RPC, RPC)],
                    deg_out.at[c, pl.ds(s * RPC, RPC)])


def _sc_edge_agg(xs_hbm, srcb_hbm, dstb_hbm, ztile_hbm, acc_out,
                 src_v, dst_v, buf0, buf1, acc_sh, sem0, sem1):
    c = lax.axis_index("c")
    s = lax.axis_index("s")
    wid = c * NS + s
    pltpu.sync_copy(ztile_hbm, acc_sh.at[pl.ds(s * RPC, RPC)])
    plsc.subcore_barrier()

    for p in range(PH):
        pltpu.sync_copy(srcb_hbm.at[wid, pl.ds(p * HC, HC)], src_v)
        pltpu.sync_copy(dstb_hbm.at[wid, pl.ds(p * HC, HC)], dst_v)

        pltpu.async_copy(xs_hbm.at[src_v.at[0]], buf0, sem0)
        pltpu.async_copy(xs_hbm.at[src_v.at[1]], buf1, sem1)

        def body(k, _):
            j0 = 2 * k
            pltpu.make_async_copy(xs_hbm.at[src_v.at[j0]], buf0, sem0).wait()
            pltpu.sync_copy(buf0, acc_sh.at[dst_v.at[j0]], add=True)
            pltpu.async_copy(xs_hbm.at[src_v.at[j0 + 2]], buf0, sem0)
            pltpu.make_async_copy(
                xs_hbm.at[src_v.at[j0 + 1]], buf1, sem1).wait()
            pltpu.sync_copy(buf1, acc_sh.at[dst_v.at[j0 + 1]], add=True)
            pltpu.async_copy(xs_hbm.at[src_v.at[j0 + 3]], buf1, sem1)
            return 0

        lax.fori_loop(0, HC // 2 - 1, body, 0)
        pltpu.make_async_copy(xs_hbm.at[src_v.at[HC - 2]], buf0, sem0).wait()
        pltpu.sync_copy(buf0, acc_sh.at[dst_v.at[HC - 2]], add=True)
        pltpu.make_async_copy(xs_hbm.at[src_v.at[HC - 1]], buf1, sem1).wait()
        pltpu.sync_copy(buf1, acc_sh.at[dst_v.at[HC - 1]], add=True)

    plsc.subcore_barrier()
    pltpu.sync_copy(acc_sh.at[pl.ds(s * RPC, RPC)],
                    acc_out.at[c, pl.ds(s * RPC, RPC)])


@functools.lru_cache(maxsize=None)
def _sc_kernels():
    mesh = plsc.VectorSubcoreMesh(core_axis_name="c", subcore_axis_name="s",
                                  num_cores=NC, num_subcores=NS)
    embed_deg = pl.kernel(
        _sc_embed_deg,
        out_type=(
            jax.ShapeDtypeStruct((NP, D), jnp.float32),
            jax.ShapeDtypeStruct((NC, NP, D), jnp.float32),
        ),
        mesh=mesh,
        scratch_types=[
            pltpu.VMEM((YC, LG), jnp.int32),
            pltpu.VMEM((EC, CL), jnp.int32),
            pltpu.VMEM((LG, D), jnp.float32),
            pltpu.VMEM((CL, D), jnp.float32),
            pltpu.VMEM_SHARED((NP, D), jnp.float32),
            pltpu.SemaphoreType.DMA,
        ],
    )
    edge_agg = pl.kernel(
        _sc_edge_agg,
        out_type=jax.ShapeDtypeStruct((NC, NP, D), jnp.float32),
        mesh=mesh,
        scratch_types=[
            pltpu.VMEM((HC, CL), jnp.int32),
            pltpu.VMEM((HC, CL), jnp.int32),
            pltpu.VMEM((CL, D), jnp.float32),
            pltpu.VMEM((CL, D), jnp.float32),
            pltpu.VMEM_SHARED((NP, D), jnp.float32),
            pltpu.SemaphoreType.DMA,
            pltpu.SemaphoreType.DMA,
        ],
    )
    return embed_deg, edge_agg


_BR = 2560


def _tc1_body(d0, d1, e, wn, bn, w1, xs1_ref, dinv_ref):
    dinv = lax.rsqrt(d0[...] + d1[...] + 1.0)
    x0 = jnp.dot(e[...], wn[...], preferred_element_type=jnp.float32) + bn[...]
    xw1 = jnp.dot(x0, w1[...], preferred_element_type=jnp.float32)
    xs1_ref[...] = dinv * xw1
    dinv_ref[...] = dinv


def _tc2_body(a0, a1, xs1, dinv, b1, w2, xs2_ref):
    h = jnp.maximum(dinv[...] * (a0[...] + a1[...] + xs1[...]) + b1[...], 0.0)
    xs2_ref[...] = dinv[...] * jnp.dot(h, w2[...],
                                       preferred_element_type=jnp.float32)


def _tc3_body(a0, a1, xs2, dinv, b2, out_ref):
    out_ref[...] = dinv[...] * (a0[...] + a1[...] + xs2[...]) + b2[...]


def _rows(shape):
    return pl.BlockSpec(shape, lambda i: (i,) + (0,) * (len(shape) - 1))


def _whole(shape):
    return pl.BlockSpec(shape, lambda i: (0,) * len(shape))


def _tc1(d0, d1, e, wn, bn, w1):
    return pl.pallas_call(
        _tc1_body,
        grid=(NP // _BR,),
        in_specs=[_rows((_BR, D)), _rows((_BR, D)), _rows((_BR, D)),
                  _whole((D, D)), _whole((1, D)), _whole((D, D))],
        out_specs=[_rows((_BR, D)), _rows((_BR, D))],
        out_shape=[jax.ShapeDtypeStruct((NP, D), jnp.float32),
                   jax.ShapeDtypeStruct((NP, D), jnp.float32)],
    )(d0, d1, e, wn, bn, w1)


def _tc2(a0, a1, xs1, dinv, b1, w2):
    return pl.pallas_call(
        _tc2_body,
        grid=(NP // _BR,),
        in_specs=[_rows((_BR, D)), _rows((_BR, D)), _rows((_BR, D)),
                  _rows((_BR, D)), _whole((1, D)), _whole((D, D))],
        out_specs=_rows((_BR, D)),
        out_shape=jax.ShapeDtypeStruct((NP, D), jnp.float32),
    )(a0, a1, xs1, dinv, b1, w2)


def _tc3(a0, a1, xs2, dinv, b2):
    return pl.pallas_call(
        _tc3_body,
        grid=(NP // _BR,),
        in_specs=[_rows((_BR, D)), _rows((_BR, D)), _rows((_BR, D)),
                  _rows((_BR, D)), _whole((1, D))],
        out_specs=_rows((_BR, D)),
        out_shape=jax.ShapeDtypeStruct((NP, D), jnp.float32),
    )(a0, a1, xs2, dinv, b2)


def kernel(y, edge_index, embed_table, W_node_w, W_node_b, W1, b1, W2, b2):
    src = edge_index[0]
    dst = edge_index[1]
    epad = NW * EPT - E
    srcb = jnp.concatenate(
        [src, jnp.zeros((epad,), jnp.int32)]).reshape(NW, EC, CL)
    dstb = jnp.concatenate(
        [dst, jnp.full((epad,), DUMP, jnp.int32)]).reshape(NW, EC, CL)
    yb = jnp.concatenate(
        [y, jnp.zeros((NP - N,), jnp.int32)]).reshape(NW, YC, LG)

    ones_rows = jnp.ones((CL, D), jnp.float32)
    ztile = jnp.zeros((RPC, D), jnp.float32)

    sc_embed_deg, sc_edge_agg = _sc_kernels()
    e_rows, deg_p = sc_embed_deg(embed_table, yb, dstb, ones_rows, ztile)
    xs1, dinv = _tc1(deg_p[0], deg_p[1], e_rows,
                     W_node_w, W_node_b.reshape(1, D), W1)
    acc1 = sc_edge_agg(xs1, srcb, dstb, ztile)
    xs2 = _tc2(acc1[0], acc1[1], xs1, dinv, b1.reshape(1, D), W2)
    acc2 = sc_edge_agg(xs2, srcb, dstb, ztile)
    out = _tc3(acc2[0], acc2[1], xs2, dinv, b2.reshape(1, D))
    return out[:N]
